# baseline (speedup 1.0000x reference)
import functools

import jax
import jax.numpy as jnp
from jax import lax
from jax.experimental import pallas as pl
from jax.experimental.pallas import tpu as pltpu
from jax.experimental.pallas import tpu_sc as plsc

N = 10000
E = 320000
NIN = 128
NHID = 128
NS = 2
LEN = 4
SSW = 0.05

NW = 32
CH = 320
NP = NW * CH
KJ = 4
KW = 80
NV = KW // 16
TB = 640
TG = NP // TB

def _wid():
    return lax.axis_index("s") * 2 + lax.axis_index("c")


def _sc_walks_body(degp, offp, dsts, u_all, hp,
                   walks1, feats0,
                   cur, pick, dgc, ofc, nxt, uv, rows,
                   sem_g, sem_w):
    wid = _wid()
    base = wid * CH
    for c in range(4):
        for j in range(KJ):
            for k in range(NV):
                cur[c, j, pl.ds(k * 16, 16)] = (
                    lax.broadcasted_iota(jnp.int32, (16,), 0)
                    + (base + j * KW + k * 16)
                )
    for t in range(LEN - 1):
        dmas = []
        for c in range(4):
            dmas.append(pltpu.make_async_copy(u_all.at[c, t, wid], uv.at[c], sem_g))
            for j in range(KJ):
                dmas.append(pltpu.make_async_copy(
                    degp.at[cur.at[c, j]], dgc.at[c, j], sem_g))
                dmas.append(pltpu.make_async_copy(
                    offp.at[cur.at[c, j]], ofc.at[c, j], sem_g))
        for d in dmas:
            d.start()
        for d in dmas:
            d.wait()
        for c in range(4):
            for j in range(KJ):
                for k in range(NV):
                    sl = pl.ds(k * 16, 16)
                    d = dgc[c, j, sl]
                    p = ofc[c, j, sl] + (
                        uv[c, j, sl] * d.astype(jnp.float32)).astype(jnp.int32)
                    pick[c, j, sl] = jnp.minimum(jnp.maximum(p, 0), E - 1)
        dmas = []
        for c in range(4):
            for j in range(KJ):
                dmas.append(pltpu.make_async_copy(
                    dsts.at[pick.at[c, j]], nxt.at[c, j], sem_g))
        for d in dmas:
            d.start()
        for d in dmas:
            d.wait()
        for c in range(4):
            for j in range(KJ):
                for k in range(NV):
                    sl = pl.ds(k * 16, 16)
                    cur[c, j, sl] = jnp.where(
                        dgc[c, j, sl] > 0, nxt[c, j, sl], cur[c, j, sl])
        gdmas = []
        for c in range(2):
            for j in range(KJ):
                gdmas.append(pltpu.make_async_copy(
                    hp.at[cur.at[c, j]], rows.at[c, pl.ds(j * KW, KW)], sem_g))
        wdmas = []
        for c in range(2, 4):
            wdmas.append(pltpu.make_async_copy(
                cur.at[c], walks1.at[(c - 2) * 3 + t, wid], sem_w))
        for d in gdmas + wdmas:
            d.start()
        for d in gdmas:
            d.wait()
        fdmas = []
        for c in range(2):
            fdmas.append(pltpu.make_async_copy(
                rows.at[c], feats0.at[c * 3 + t, pl.ds(base, CH)], sem_w))
        for d in fdmas:
            d.start()
        for d in wdmas + fdmas:
            d.wait()


@functools.cache
def _get_sc_walks():
  return pl.kernel(
    _sc_walks_body,
    out_type=[
        jax.ShapeDtypeStruct((6, NW, KJ, KW), jnp.int32),
        jax.ShapeDtypeStruct((6, NP, NIN), jnp.float32),
    ],
    mesh=plsc.VectorSubcoreMesh(core_axis_name="c", subcore_axis_name="s"),
    scratch_types=[
        pltpu.VMEM((4, KJ, KW), jnp.int32),
        pltpu.VMEM((4, KJ, KW), jnp.int32),
        pltpu.VMEM((4, KJ, KW), jnp.int32),
        pltpu.VMEM((4, KJ, KW), jnp.int32),
        pltpu.VMEM((4, KJ, KW), jnp.int32),
        pltpu.VMEM((4, KJ, KW), jnp.float32),
        pltpu.VMEM((2, CH, NIN), jnp.float32),
        pltpu.SemaphoreType.DMA,
        pltpu.SemaphoreType.DMA,
    ],
  )


def _sc_gather_body(x1p, walks1, feats1, idxv, rows, sem_g, sem_w):
    wid = _wid()
    base = wid * CH
    for c6 in range(6):
        ld = pltpu.make_async_copy(walks1.at[c6, wid], idxv, sem_g)
        ld.start()
        ld.wait()
        dmas = []
        for j in range(KJ):
            dmas.append(pltpu.make_async_copy(
                x1p.at[idxv.at[j]], rows.at[pl.ds(j * KW, KW)], sem_g))
        for d in dmas:
            d.start()
        for d in dmas:
            d.wait()
        w = pltpu.make_async_copy(rows, feats1.at[c6, pl.ds(base, CH)], sem_w)
        w.start()
        w.wait()


@functools.cache
def _get_sc_gather():
  return pl.kernel(
    _sc_gather_body,
    out_type=[jax.ShapeDtypeStruct((6, NP, NHID), jnp.float32)],
    mesh=plsc.VectorSubcoreMesh(core_axis_name="c", subcore_axis_name="s"),
    scratch_types=[
        pltpu.VMEM((KJ, KW), jnp.int32),
        pltpu.VMEM((CH, NHID), jnp.float32),
        pltpu.SemaphoreType.DMA,
        pltpu.SemaphoreType.DMA,
    ],
  )


def _elu(x):
    return jnp.where(x > 0, x, jnp.exp(jnp.minimum(x, 0.0)) - 1.0)


def _gru_block(x0, f_ref, s, Wi, bi, Wh, bh):
    h = jnp.zeros((TB, NHID), jnp.float32)
    for t in range(LEN):
        xt = x0 if t == 0 else f_ref[s * 3 + (t - 1)]
        gi = jnp.dot(xt, Wi, preferred_element_type=jnp.float32) + bi
        gh = jnp.dot(h, Wh, preferred_element_type=jnp.float32) + bh
        r = jax.nn.sigmoid(gi[:, :NHID] + gh[:, :NHID])
        z = jax.nn.sigmoid(gi[:, NHID:2 * NHID] + gh[:, NHID:2 * NHID])
        n = jnp.tanh(gi[:, 2 * NHID:] + r * gh[:, 2 * NHID:])
        h = (1.0 - z) * n + z * h
    return h


def _enc_body(h_ref, Wenc_ref, benc_ref, x_ref):
    x_ref[...] = jnp.dot(h_ref[...], Wenc_ref[...],
                         preferred_element_type=jnp.float32) + benc_ref[...]


def _l0_body(xp_ref, f_ref, hp_ref, Wi_ref, bi_ref, Wh_ref,
             bh_ref, Wss_ref, bss_ref,
             x1_ref, l0_ref,
             acc_ref):
    i = pl.program_id(0)

    @pl.when(i == 0)
    def _():
        acc_ref[0, 0] = 0.0

    xblk = xp_ref[...]
    hblk = hp_ref[...]
    rid = lax.broadcasted_iota(jnp.int32, (TB, 1), 0) + i * TB
    maskf = (rid < N).astype(jnp.float32)
    xacc = jnp.zeros((TB, NHID), jnp.float32)
    ssacc = jnp.float32(0.0)
    for s in range(NS):
        hfin = _gru_block(xblk, f_ref, s, Wi_ref[...], bi_ref[...],
                          Wh_ref[...], bh_ref[...])
        o = _elu(hfin)
        xacc = xacc + o * jax.nn.sigmoid(o)
        pred = jnp.dot(hfin, Wss_ref[...],
                       preferred_element_type=jnp.float32) + bss_ref[...]
        dd = (pred - hblk) * maskf
        ssacc = ssacc + jnp.sum(dd * dd)
    x1_ref[...] = xacc * (1.0 / NS)
    acc_ref[0, 0] += ssacc

    @pl.when(i == TG - 1)
    def _():
        l0_ref[...] = jnp.reshape(
            acc_ref[0, 0] * (1.0 / (NS * N * NIN)), (1, 1))


def _l1_body(x1_ref, f_ref, hp_ref, Wi_ref, bi_ref, Wh_ref, bh_ref,
             Wss_ref, bss_ref, Wd1_ref, bd1_ref, Wd2_ref, bd2_ref,
             y_ref, l1_ref,
             gsum_ref, acc_ref):
    i = pl.program_id(0)

    @pl.when(i == 0)
    def _():
        gsum_ref[...] = jnp.zeros((1, NHID), jnp.float32)
        acc_ref[0, 0] = 0.0

    xblk = x1_ref[...]
    hblk = hp_ref[...]
    rid = lax.broadcasted_iota(jnp.int32, (TB, 1), 0) + i * TB
    maskf = (rid < N).astype(jnp.float32)
    hfacc = jnp.zeros((TB, NHID), jnp.float32)
    ssacc = jnp.float32(0.0)
    for s in range(NS):
        hfin = _gru_block(xblk, f_ref, s, Wi_ref[...], bi_ref[...],
                          Wh_ref[...], bh_ref[...])
        hfacc = hfacc + _elu(hfin)
        pred = jnp.dot(hfin, Wss_ref[...],
                       preferred_element_type=jnp.float32) + bss_ref[...]
        dd = (pred - hblk) * maskf
        ssacc = ssacc + jnp.sum(dd * dd)
    acc_ref[0, 0] += ssacc
    gsum_ref[...] += jnp.sum(hfacc * (0.5 * maskf), axis=0, keepdims=True)

    @pl.when(i == TG - 1)
    def _():
        g = gsum_ref[...] * (1.0 / N)
        t1 = jnp.dot(_elu(g), Wd1_ref[...],
                     preferred_element_type=jnp.float32) + bd1_ref[...]
        y_ref[...] = jnp.dot(_elu(t1), Wd2_ref[...],
                             preferred_element_type=jnp.float32) + bd2_ref[...]
        l1_ref[...] = jnp.reshape(
            acc_ref[0, 0] * (1.0 / (NS * N * NIN)), (1, 1))


def _full_spec(shape):
    return pl.BlockSpec(shape, lambda i: tuple(0 for _ in shape))


_enc_in_specs = [
    pl.BlockSpec((TB, NIN), lambda i: (i, 0)),
    _full_spec((NIN, NHID)),
    _full_spec((1, NHID)),
]
_enc_out_specs = [pl.BlockSpec((TB, NHID), lambda i: (i, 0))]
_enc_out_shape = [jax.ShapeDtypeStruct((NP, NHID), jnp.float32)]

_tc_enc = pl.pallas_call(
    _enc_body,
    grid=(TG,),
    in_specs=_enc_in_specs,
    out_specs=_enc_out_specs[0],
    out_shape=_enc_out_shape[0],
)

_tc0_in_specs = [
    pl.BlockSpec((TB, NHID), lambda i: (i, 0)),
    pl.BlockSpec((6, TB, NIN), lambda i: (0, i, 0)),
    pl.BlockSpec((TB, NIN), lambda i: (i, 0)),
    _full_spec((NHID, 3 * NHID)),
    _full_spec((1, 3 * NHID)),
    _full_spec((NHID, 3 * NHID)),
    _full_spec((1, 3 * NHID)),
    _full_spec((NHID, NIN)),
    _full_spec((1, NIN)),
]
_tc0_out_specs = [
    pl.BlockSpec((TB, NHID), lambda i: (i, 0)),
    pl.BlockSpec((1, 1), lambda i: (0, 0)),
]
_tc0_out_shape = [
    jax.ShapeDtypeStruct((NP, NHID), jnp.float32),
    jax.ShapeDtypeStruct((1, 1), jnp.float32),
]
_tc0_scratch = [
    pltpu.SMEM((1, 1), jnp.float32),
]

_tc_layer0 = pl.pallas_call(
    _l0_body,
    grid=(TG,),
    in_specs=_tc0_in_specs,
    out_specs=_tc0_out_specs,
    out_shape=_tc0_out_shape,
    scratch_shapes=_tc0_scratch,
)

_tc1_in_specs = [
    pl.BlockSpec((TB, NHID), lambda i: (i, 0)),
    pl.BlockSpec((6, TB, NHID), lambda i: (0, i, 0)),
    pl.BlockSpec((TB, NIN), lambda i: (i, 0)),
    _full_spec((NHID, 3 * NHID)),
    _full_spec((1, 3 * NHID)),
    _full_spec((NHID, 3 * NHID)),
    _full_spec((1, 3 * NHID)),
    _full_spec((NHID, NIN)),
    _full_spec((1, NIN)),
    _full_spec((NHID, NHID)),
    _full_spec((1, NHID)),
    _full_spec((NHID, 1)),
    _full_spec((1, 1)),
]
_tc1_out_specs = [
    pl.BlockSpec((1, 1), lambda i: (0, 0)),
    pl.BlockSpec((1, 1), lambda i: (0, 0)),
]
_tc1_out_shape = [
    jax.ShapeDtypeStruct((1, 1), jnp.float32),
    jax.ShapeDtypeStruct((1, 1), jnp.float32),
]
_tc1_scratch = [
    pltpu.VMEM((1, NHID), jnp.float32),
    pltpu.SMEM((1, 1), jnp.float32),
]

_tc_layer1 = pl.pallas_call(
    _l1_body,
    grid=(TG,),
    in_specs=_tc1_in_specs,
    out_specs=_tc1_out_specs,
    out_shape=_tc1_out_shape,
    scratch_shapes=_tc1_scratch,
)


def kernel(h, edge_index, W_enc, b_enc, Wi0, Wh0, bi0, bh0, Wss0, bss0,
           Wi1, Wh1, bi1, bh1, Wss1, bss1, Wd1, bd1, Wd2, bd2):
    src = edge_index[0]
    dst = edge_index[1]
    order = jnp.argsort(src)
    dst_sorted = dst[order].astype(jnp.int32)
    deg = jnp.bincount(src, length=N).astype(jnp.int32)
    offsets = (jnp.cumsum(deg) - deg).astype(jnp.int32)
    degp = jnp.pad(deg, (0, NP - N))
    offp = jnp.pad(offsets, (0, NP - N))

    wkey = jax.random.key(42)
    us = []
    for layer in range(2):
        lk = jax.random.fold_in(wkey, layer)
        for s in range(NS):
            for t in range(LEN - 1):
                us.append(jax.random.uniform(
                    jax.random.fold_in(lk, s * 97 + t), (N,)))
    u_all = jnp.stack(us).reshape(4, LEN - 1, N)
    u_all = jnp.pad(u_all, ((0, 0), (0, 0), (0, NP - N)))
    u_all = u_all.reshape(4, LEN - 1, NW, KJ, KW)

    hp = jnp.pad(h, ((0, NP - N), (0, 0)))
    xp = _tc_enc(hp, W_enc, b_enc.reshape(1, NHID))

    walks1, feats0 = _get_sc_walks()(degp, offp, dst_sorted, u_all, xp)

    bi02 = bi0.reshape(1, 3 * NHID)
    bh02 = bh0.reshape(1, 3 * NHID)
    bss02 = bss0.reshape(1, NIN)
    x1p, l0 = _tc_layer0(xp, feats0, hp, Wi0, bi02, Wh0, bh02,
                         Wss0, bss02)

    (feats1,) = _get_sc_gather()(x1p, walks1)

    bi12 = bi1.reshape(1, 3 * NHID)
    bh12 = bh1.reshape(1, 3 * NHID)
    bss12 = bss1.reshape(1, NIN)
    bd12 = bd1.reshape(1, NHID)
    bd22 = bd2.reshape(1, 1)
    y, l1 = _tc_layer1(x1p, feats1, hp, Wi1, bi12, Wh1, bh12, Wss1, bss12,
                       Wd1, bd12, Wd2, bd22)

    loss = SSW * (l0[0, 0] + l1[0, 0])
    return (y, loss)


# speedup vs baseline: 1.0221x; 1.0221x over previous
import functools

import jax
import jax.numpy as jnp
from jax import lax
from jax.experimental import pallas as pl
from jax.experimental.pallas import tpu as pltpu
from jax.experimental.pallas import tpu_sc as plsc

N = 10000
E = 320000
NIN = 128
NHID = 128
NS = 2
LEN = 4
SSW = 0.05

NW = 32
CH = 320
NP = NW * CH
KJ = 4
KW = 80
NV = KW // 16
TB = 1280
TG = NP // TB

def _wid():
    return lax.axis_index("s") * 2 + lax.axis_index("c")


def _sc_walks_body(degp, offp, dsts, u_all, hp,
                   walks1, feats0,
                   cur, pick, dgc, ofc, nxt, uv, rows,
                   sem_g, sem_w, sem_f):
    wid = _wid()
    base = wid * CH
    for c in range(4):
        for j in range(KJ):
            for k in range(NV):
                cur[c, j, pl.ds(k * 16, 16)] = (
                    lax.broadcasted_iota(jnp.int32, (16,), 0)
                    + (base + j * KW + k * 16)
                )
    prev_w = []
    prev_f = []
    for t in range(LEN - 1):
        dmas = []
        for c in range(4):
            dmas.append(pltpu.make_async_copy(u_all.at[c, t, wid], uv.at[c], sem_g))
            for j in range(KJ):
                dmas.append(pltpu.make_async_copy(
                    degp.at[cur.at[c, j]], dgc.at[c, j], sem_g))
                dmas.append(pltpu.make_async_copy(
                    offp.at[cur.at[c, j]], ofc.at[c, j], sem_g))
        for d in dmas:
            d.start()
        for d in dmas:
            d.wait()
        for c in range(4):
            for j in range(KJ):
                for k in range(NV):
                    sl = pl.ds(k * 16, 16)
                    d = dgc[c, j, sl]
                    p = ofc[c, j, sl] + (
                        uv[c, j, sl] * d.astype(jnp.float32)).astype(jnp.int32)
                    pick[c, j, sl] = jnp.minimum(jnp.maximum(p, 0), E - 1)
        dmas = []
        for c in range(4):
            for j in range(KJ):
                dmas.append(pltpu.make_async_copy(
                    dsts.at[pick.at[c, j]], nxt.at[c, j], sem_g))
        for d in dmas:
            d.start()
        for d in dmas:
            d.wait()
        for d in prev_w:
            d.wait()
        for c in range(4):
            for j in range(KJ):
                for k in range(NV):
                    sl = pl.ds(k * 16, 16)
                    cur[c, j, sl] = jnp.where(
                        dgc[c, j, sl] > 0, nxt[c, j, sl], cur[c, j, sl])
        for d in prev_f:
            d.wait()
        gdmas = []
        for c in range(2):
            for j in range(KJ):
                gdmas.append(pltpu.make_async_copy(
                    hp.at[cur.at[c, j]], rows.at[c, pl.ds(j * KW, KW)], sem_g))
        wdmas = []
        for c in range(2, 4):
            wdmas.append(pltpu.make_async_copy(
                cur.at[c], walks1.at[(c - 2) * 3 + t, wid], sem_w))
        for d in gdmas + wdmas:
            d.start()
        for d in gdmas:
            d.wait()
        fdmas = []
        for c in range(2):
            fdmas.append(pltpu.make_async_copy(
                rows.at[c], feats0.at[c * 3 + t, pl.ds(base, CH)], sem_f))
        for d in fdmas:
            d.start()
        prev_w = wdmas
        prev_f = fdmas
    for d in prev_w + prev_f:
        d.wait()


@functools.cache
def _get_sc_walks():
  return pl.kernel(
    _sc_walks_body,
    out_type=[
        jax.ShapeDtypeStruct((6, NW, KJ, KW), jnp.int32),
        jax.ShapeDtypeStruct((6, NP, NIN), jnp.float32),
    ],
    mesh=plsc.VectorSubcoreMesh(core_axis_name="c", subcore_axis_name="s"),
    scratch_types=[
        pltpu.VMEM((4, KJ, KW), jnp.int32),
        pltpu.VMEM((4, KJ, KW), jnp.int32),
        pltpu.VMEM((4, KJ, KW), jnp.int32),
        pltpu.VMEM((4, KJ, KW), jnp.int32),
        pltpu.VMEM((4, KJ, KW), jnp.int32),
        pltpu.VMEM((4, KJ, KW), jnp.float32),
        pltpu.VMEM((2, CH, NIN), jnp.float32),
        pltpu.SemaphoreType.DMA,
        pltpu.SemaphoreType.DMA,
        pltpu.SemaphoreType.DMA,
    ],
  )


def _sc_gather_body(x1p, walks1, feats1, idxv, rows, sem_g, sem_w):
    wid = _wid()
    base = wid * CH
    for c6 in range(6):
        ld = pltpu.make_async_copy(walks1.at[c6, wid], idxv, sem_g)
        ld.start()
        ld.wait()
        dmas = []
        for j in range(KJ):
            dmas.append(pltpu.make_async_copy(
                x1p.at[idxv.at[j]], rows.at[pl.ds(j * KW, KW)], sem_g))
        for d in dmas:
            d.start()
        for d in dmas:
            d.wait()
        w = pltpu.make_async_copy(rows, feats1.at[c6, pl.ds(base, CH)], sem_w)
        w.start()
        w.wait()


@functools.cache
def _get_sc_gather():
  return pl.kernel(
    _sc_gather_body,
    out_type=[jax.ShapeDtypeStruct((6, NP, NHID), jnp.float32)],
    mesh=plsc.VectorSubcoreMesh(core_axis_name="c", subcore_axis_name="s"),
    scratch_types=[
        pltpu.VMEM((KJ, KW), jnp.int32),
        pltpu.VMEM((CH, NHID), jnp.float32),
        pltpu.SemaphoreType.DMA,
        pltpu.SemaphoreType.DMA,
    ],
  )


def _elu(x):
    return jnp.where(x > 0, x, jnp.exp(jnp.minimum(x, 0.0)) - 1.0)


def _gru_block(x0, f_ref, s, Wi, bi, Wh, bh):
    h = jnp.zeros((TB, NHID), jnp.float32)
    for t in range(LEN):
        xt = x0 if t == 0 else f_ref[s * 3 + (t - 1)]
        gi = jnp.dot(xt, Wi, preferred_element_type=jnp.float32) + bi
        gh = jnp.dot(h, Wh, preferred_element_type=jnp.float32) + bh
        r = jax.nn.sigmoid(gi[:, :NHID] + gh[:, :NHID])
        z = jax.nn.sigmoid(gi[:, NHID:2 * NHID] + gh[:, NHID:2 * NHID])
        n = jnp.tanh(gi[:, 2 * NHID:] + r * gh[:, 2 * NHID:])
        h = (1.0 - z) * n + z * h
    return h


def _enc_body(h_ref, Wenc_ref, benc_ref, x_ref):
    x_ref[...] = jnp.dot(h_ref[...], Wenc_ref[...],
                         preferred_element_type=jnp.float32) + benc_ref[...]


def _l0_body(xp_ref, f_ref, hp_ref, Wi_ref, bi_ref, Wh_ref,
             bh_ref, Wss_ref, bss_ref,
             x1_ref, l0_ref,
             acc_ref):
    i = pl.program_id(0)

    @pl.when(i == 0)
    def _():
        acc_ref[0, 0] = 0.0

    xblk = xp_ref[...]
    hblk = hp_ref[...]
    rid = lax.broadcasted_iota(jnp.int32, (TB, 1), 0) + i * TB
    maskf = (rid < N).astype(jnp.float32)
    xacc = jnp.zeros((TB, NHID), jnp.float32)
    ssacc = jnp.float32(0.0)
    for s in range(NS):
        hfin = _gru_block(xblk, f_ref, s, Wi_ref[...], bi_ref[...],
                          Wh_ref[...], bh_ref[...])
        o = _elu(hfin)
        xacc = xacc + o * jax.nn.sigmoid(o)
        pred = jnp.dot(hfin, Wss_ref[...],
                       preferred_element_type=jnp.float32) + bss_ref[...]
        dd = (pred - hblk) * maskf
        ssacc = ssacc + jnp.sum(dd * dd)
    x1_ref[...] = xacc * (1.0 / NS)
    acc_ref[0, 0] += ssacc

    @pl.when(i == TG - 1)
    def _():
        l0_ref[...] = jnp.reshape(
            acc_ref[0, 0] * (1.0 / (NS * N * NIN)), (1, 1))


def _l1_body(x1_ref, f_ref, hp_ref, Wi_ref, bi_ref, Wh_ref, bh_ref,
             Wss_ref, bss_ref, Wd1_ref, bd1_ref, Wd2_ref, bd2_ref,
             y_ref, l1_ref,
             gsum_ref, acc_ref):
    i = pl.program_id(0)

    @pl.when(i == 0)
    def _():
        gsum_ref[...] = jnp.zeros((1, NHID), jnp.float32)
        acc_ref[0, 0] = 0.0

    xblk = x1_ref[...]
    hblk = hp_ref[...]
    rid = lax.broadcasted_iota(jnp.int32, (TB, 1), 0) + i * TB
    maskf = (rid < N).astype(jnp.float32)
    hfacc = jnp.zeros((TB, NHID), jnp.float32)
    ssacc = jnp.float32(0.0)
    for s in range(NS):
        hfin = _gru_block(xblk, f_ref, s, Wi_ref[...], bi_ref[...],
                          Wh_ref[...], bh_ref[...])
        hfacc = hfacc + _elu(hfin)
        pred = jnp.dot(hfin, Wss_ref[...],
                       preferred_element_type=jnp.float32) + bss_ref[...]
        dd = (pred - hblk) * maskf
        ssacc = ssacc + jnp.sum(dd * dd)
    acc_ref[0, 0] += ssacc
    gsum_ref[...] += jnp.sum(hfacc * (0.5 * maskf), axis=0, keepdims=True)

    @pl.when(i == TG - 1)
    def _():
        g = gsum_ref[...] * (1.0 / N)
        t1 = jnp.dot(_elu(g), Wd1_ref[...],
                     preferred_element_type=jnp.float32) + bd1_ref[...]
        y_ref[...] = jnp.dot(_elu(t1), Wd2_ref[...],
                             preferred_element_type=jnp.float32) + bd2_ref[...]
        l1_ref[...] = jnp.reshape(
            acc_ref[0, 0] * (1.0 / (NS * N * NIN)), (1, 1))


def _full_spec(shape):
    return pl.BlockSpec(shape, lambda i: tuple(0 for _ in shape))


_enc_in_specs = [
    pl.BlockSpec((TB, NIN), lambda i: (i, 0)),
    _full_spec((NIN, NHID)),
    _full_spec((1, NHID)),
]
_enc_out_specs = [pl.BlockSpec((TB, NHID), lambda i: (i, 0))]
_enc_out_shape = [jax.ShapeDtypeStruct((NP, NHID), jnp.float32)]

_tc_enc = pl.pallas_call(
    _enc_body,
    grid=(TG,),
    in_specs=_enc_in_specs,
    out_specs=_enc_out_specs[0],
    out_shape=_enc_out_shape[0],
)

_tc0_in_specs = [
    pl.BlockSpec((TB, NHID), lambda i: (i, 0)),
    pl.BlockSpec((6, TB, NIN), lambda i: (0, i, 0)),
    pl.BlockSpec((TB, NIN), lambda i: (i, 0)),
    _full_spec((NHID, 3 * NHID)),
    _full_spec((1, 3 * NHID)),
    _full_spec((NHID, 3 * NHID)),
    _full_spec((1, 3 * NHID)),
    _full_spec((NHID, NIN)),
    _full_spec((1, NIN)),
]
_tc0_out_specs = [
    pl.BlockSpec((TB, NHID), lambda i: (i, 0)),
    pl.BlockSpec((1, 1), lambda i: (0, 0)),
]
_tc0_out_shape = [
    jax.ShapeDtypeStruct((NP, NHID), jnp.float32),
    jax.ShapeDtypeStruct((1, 1), jnp.float32),
]
_tc0_scratch = [
    pltpu.SMEM((1, 1), jnp.float32),
]

_tc_layer0 = pl.pallas_call(
    _l0_body,
    grid=(TG,),
    in_specs=_tc0_in_specs,
    out_specs=_tc0_out_specs,
    out_shape=_tc0_out_shape,
    scratch_shapes=_tc0_scratch,
)

_tc1_in_specs = [
    pl.BlockSpec((TB, NHID), lambda i: (i, 0)),
    pl.BlockSpec((6, TB, NHID), lambda i: (0, i, 0)),
    pl.BlockSpec((TB, NIN), lambda i: (i, 0)),
    _full_spec((NHID, 3 * NHID)),
    _full_spec((1, 3 * NHID)),
    _full_spec((NHID, 3 * NHID)),
    _full_spec((1, 3 * NHID)),
    _full_spec((NHID, NIN)),
    _full_spec((1, NIN)),
    _full_spec((NHID, NHID)),
    _full_spec((1, NHID)),
    _full_spec((NHID, 1)),
    _full_spec((1, 1)),
]
_tc1_out_specs = [
    pl.BlockSpec((1, 1), lambda i: (0, 0)),
    pl.BlockSpec((1, 1), lambda i: (0, 0)),
]
_tc1_out_shape = [
    jax.ShapeDtypeStruct((1, 1), jnp.float32),
    jax.ShapeDtypeStruct((1, 1), jnp.float32),
]
_tc1_scratch = [
    pltpu.VMEM((1, NHID), jnp.float32),
    pltpu.SMEM((1, 1), jnp.float32),
]

_tc_layer1 = pl.pallas_call(
    _l1_body,
    grid=(TG,),
    in_specs=_tc1_in_specs,
    out_specs=_tc1_out_specs,
    out_shape=_tc1_out_shape,
    scratch_shapes=_tc1_scratch,
)


def kernel(h, edge_index, W_enc, b_enc, Wi0, Wh0, bi0, bh0, Wss0, bss0,
           Wi1, Wh1, bi1, bh1, Wss1, bss1, Wd1, bd1, Wd2, bd2):
    src = edge_index[0]
    dst = edge_index[1]
    _, dst_sorted = lax.sort_key_val(src, dst, is_stable=True)
    dst_sorted = dst_sorted.astype(jnp.int32)
    deg = jnp.bincount(src, length=N).astype(jnp.int32)
    offsets = (jnp.cumsum(deg) - deg).astype(jnp.int32)
    degp = jnp.pad(deg, (0, NP - N))
    offp = jnp.pad(offsets, (0, NP - N))

    wkey = jax.random.key(42)
    us = []
    for layer in range(2):
        lk = jax.random.fold_in(wkey, layer)
        for s in range(NS):
            for t in range(LEN - 1):
                us.append(jax.random.uniform(
                    jax.random.fold_in(lk, s * 97 + t), (N,)))
    u_all = jnp.stack(us).reshape(4, LEN - 1, N)
    u_all = jnp.pad(u_all, ((0, 0), (0, 0), (0, NP - N)))
    u_all = u_all.reshape(4, LEN - 1, NW, KJ, KW)

    hp = jnp.pad(h, ((0, NP - N), (0, 0)))
    xp = _tc_enc(hp, W_enc, b_enc.reshape(1, NHID))

    walks1, feats0 = _get_sc_walks()(degp, offp, dst_sorted, u_all, xp)

    bi02 = bi0.reshape(1, 3 * NHID)
    bh02 = bh0.reshape(1, 3 * NHID)
    bss02 = bss0.reshape(1, NIN)
    x1p, l0 = _tc_layer0(xp, feats0, hp, Wi0, bi02, Wh0, bh02,
                         Wss0, bss02)

    (feats1,) = _get_sc_gather()(x1p, walks1)

    bi12 = bi1.reshape(1, 3 * NHID)
    bh12 = bh1.reshape(1, 3 * NHID)
    bss12 = bss1.reshape(1, NIN)
    bd12 = bd1.reshape(1, NHID)
    bd22 = bd2.reshape(1, 1)
    y, l1 = _tc_layer1(x1p, feats1, hp, Wi1, bi12, Wh1, bh12, Wss1, bss12,
                       Wd1, bd12, Wd2, bd22)

    loss = SSW * (l0[0, 0] + l1[0, 0])
    return (y, loss)


# speedup vs baseline: 1.0969x; 1.0732x over previous
import functools

import jax
import jax.numpy as jnp
from jax import lax
from jax.experimental import pallas as pl
from jax.experimental.pallas import tpu as pltpu
from jax.experimental.pallas import tpu_sc as plsc

N = 10000
E = 320000
NIN = 128
NHID = 128
NS = 2
LEN = 4
SSW = 0.05

NW = 32
CH = 320
NP = NW * CH
KJ = 4
KW = 80
NV = KW // 16
TB = 1280
TG = NP // TB

def _wid():
    return lax.axis_index("s") * 2 + lax.axis_index("c")


def _sc_walks_body(degp, offp, dsts, u_all, hp,
                   walks1, feats0,
                   cur, pick, dgc, ofc, nxt, uv, rows,
                   sem_g, sem_w, sem_f):
    wid = _wid()
    base = wid * CH
    for c in range(4):
        for j in range(KJ):
            for k in range(NV):
                cur[c, j, pl.ds(k * 16, 16)] = (
                    lax.broadcasted_iota(jnp.int32, (16,), 0)
                    + (base + j * KW + k * 16)
                )
    prev_w = []
    prev_f = []
    for t in range(LEN - 1):
        dmas = []
        for c in range(4):
            dmas.append(pltpu.make_async_copy(u_all.at[c, t, wid], uv.at[c], sem_g))
            for j in range(KJ):
                dmas.append(pltpu.make_async_copy(
                    degp.at[cur.at[c, j]], dgc.at[c, j], sem_g))
                dmas.append(pltpu.make_async_copy(
                    offp.at[cur.at[c, j]], ofc.at[c, j], sem_g))
        for d in dmas:
            d.start()
        for d in dmas:
            d.wait()
        for c in range(4):
            for j in range(KJ):
                for k in range(NV):
                    sl = pl.ds(k * 16, 16)
                    d = dgc[c, j, sl]
                    p = ofc[c, j, sl] + (
                        uv[c, j, sl] * d.astype(jnp.float32)).astype(jnp.int32)
                    pick[c, j, sl] = jnp.minimum(jnp.maximum(p, 0), E - 1)
        dmas = []
        for c in range(4):
            for j in range(KJ):
                dmas.append(pltpu.make_async_copy(
                    dsts.at[pick.at[c, j]], nxt.at[c, j], sem_g))
        for d in dmas:
            d.start()
        for d in dmas:
            d.wait()
        for d in prev_w:
            d.wait()
        for c in range(4):
            for j in range(KJ):
                for k in range(NV):
                    sl = pl.ds(k * 16, 16)
                    cur[c, j, sl] = jnp.where(
                        dgc[c, j, sl] > 0, nxt[c, j, sl], cur[c, j, sl])
        for d in prev_f:
            d.wait()
        gdmas = []
        for c in range(2):
            for j in range(KJ):
                gdmas.append(pltpu.make_async_copy(
                    hp.at[cur.at[c, j]], rows.at[c, pl.ds(j * KW, KW)], sem_g))
        wdmas = []
        for c in range(2, 4):
            wdmas.append(pltpu.make_async_copy(
                cur.at[c], walks1.at[(c - 2) * 3 + t, wid], sem_w))
        for d in gdmas + wdmas:
            d.start()
        for d in gdmas:
            d.wait()
        fdmas = []
        for c in range(2):
            fdmas.append(pltpu.make_async_copy(
                rows.at[c], feats0.at[c * 3 + t, pl.ds(base, CH)], sem_f))
        for d in fdmas:
            d.start()
        prev_w = wdmas
        prev_f = fdmas
    for d in prev_w + prev_f:
        d.wait()


@functools.cache
def _get_sc_walks():
  return pl.kernel(
    _sc_walks_body,
    out_type=[
        jax.ShapeDtypeStruct((6, NW, KJ, KW), jnp.int32),
        jax.ShapeDtypeStruct((6, NP, NIN), jnp.float32),
    ],
    mesh=plsc.VectorSubcoreMesh(core_axis_name="c", subcore_axis_name="s"),
    scratch_types=[
        pltpu.VMEM((4, KJ, KW), jnp.int32),
        pltpu.VMEM((4, KJ, KW), jnp.int32),
        pltpu.VMEM((4, KJ, KW), jnp.int32),
        pltpu.VMEM((4, KJ, KW), jnp.int32),
        pltpu.VMEM((4, KJ, KW), jnp.int32),
        pltpu.VMEM((4, KJ, KW), jnp.float32),
        pltpu.VMEM((2, CH, NIN), jnp.float32),
        pltpu.SemaphoreType.DMA,
        pltpu.SemaphoreType.DMA,
        pltpu.SemaphoreType.DMA,
    ],
  )


def _sc_gather_body(x1p, walks1, feats1, idxv, rows, sem_g, sem_w):
    wid = _wid()
    base = wid * CH
    lds = [pltpu.make_async_copy(walks1.at[c6, wid], idxv.at[c6], sem_g)
           for c6 in range(6)]
    for d in lds:
        d.start()
    for d in lds:
        d.wait()
    prev_w = [None, None]
    for c6 in range(6):
        b = c6 % 2
        if prev_w[b] is not None:
            prev_w[b].wait()
        dmas = []
        for j in range(KJ):
            dmas.append(pltpu.make_async_copy(
                x1p.at[idxv.at[c6, j]], rows.at[b, pl.ds(j * KW, KW)], sem_g))
        for d in dmas:
            d.start()
        for d in dmas:
            d.wait()
        w = pltpu.make_async_copy(
            rows.at[b], feats1.at[c6, pl.ds(base, CH)], sem_w)
        w.start()
        prev_w[b] = w
    for w in prev_w:
        w.wait()


@functools.cache
def _get_sc_gather():
  return pl.kernel(
    _sc_gather_body,
    out_type=[jax.ShapeDtypeStruct((6, NP, NHID), jnp.float32)],
    mesh=plsc.VectorSubcoreMesh(core_axis_name="c", subcore_axis_name="s"),
    scratch_types=[
        pltpu.VMEM((6, KJ, KW), jnp.int32),
        pltpu.VMEM((2, CH, NHID), jnp.float32),
        pltpu.SemaphoreType.DMA,
        pltpu.SemaphoreType.DMA,
    ],
  )


def _elu(x):
    return jnp.where(x > 0, x, jnp.exp(jnp.minimum(x, 0.0)) - 1.0)


def _gru_block(x0, f_ref, s, Wi, bi, Wh, bh):
    h = jnp.zeros((TB, NHID), jnp.float32)
    for t in range(LEN):
        xt = x0 if t == 0 else f_ref[s * 3 + (t - 1)]
        gi = jnp.dot(xt, Wi, preferred_element_type=jnp.float32) + bi
        gh = jnp.dot(h, Wh, preferred_element_type=jnp.float32) + bh
        r = jax.nn.sigmoid(gi[:, :NHID] + gh[:, :NHID])
        z = jax.nn.sigmoid(gi[:, NHID:2 * NHID] + gh[:, NHID:2 * NHID])
        n = jnp.tanh(gi[:, 2 * NHID:] + r * gh[:, 2 * NHID:])
        h = (1.0 - z) * n + z * h
    return h


def _enc_body(h_ref, Wenc_ref, benc_ref, x_ref):
    x_ref[...] = jnp.dot(h_ref[...], Wenc_ref[...],
                         preferred_element_type=jnp.float32) + benc_ref[...]


def _l0_body(xp_ref, f_ref, hp_ref, Wi_ref, bi_ref, Wh_ref,
             bh_ref, Wss_ref, bss_ref,
             x1_ref, l0_ref,
             acc_ref):
    i = pl.program_id(0)

    @pl.when(i == 0)
    def _():
        acc_ref[0, 0] = 0.0

    xblk = xp_ref[...]
    hblk = hp_ref[...]
    rid = lax.broadcasted_iota(jnp.int32, (TB, 1), 0) + i * TB
    maskf = (rid < N).astype(jnp.float32)
    xacc = jnp.zeros((TB, NHID), jnp.float32)
    ssacc = jnp.float32(0.0)
    for s in range(NS):
        hfin = _gru_block(xblk, f_ref, s, Wi_ref[...], bi_ref[...],
                          Wh_ref[...], bh_ref[...])
        o = _elu(hfin)
        xacc = xacc + o * jax.nn.sigmoid(o)
        pred = jnp.dot(hfin, Wss_ref[...],
                       preferred_element_type=jnp.float32) + bss_ref[...]
        dd = (pred - hblk) * maskf
        ssacc = ssacc + jnp.sum(dd * dd)
    x1_ref[...] = xacc * (1.0 / NS)
    acc_ref[0, 0] += ssacc

    @pl.when(i == TG - 1)
    def _():
        l0_ref[...] = jnp.reshape(
            acc_ref[0, 0] * (1.0 / (NS * N * NIN)), (1, 1))


def _l1_body(x1_ref, f_ref, hp_ref, Wi_ref, bi_ref, Wh_ref, bh_ref,
             Wss_ref, bss_ref, Wd1_ref, bd1_ref, Wd2_ref, bd2_ref,
             y_ref, l1_ref,
             gsum_ref, acc_ref):
    i = pl.program_id(0)

    @pl.when(i == 0)
    def _():
        gsum_ref[...] = jnp.zeros((1, NHID), jnp.float32)
        acc_ref[0, 0] = 0.0

    xblk = x1_ref[...]
    hblk = hp_ref[...]
    rid = lax.broadcasted_iota(jnp.int32, (TB, 1), 0) + i * TB
    maskf = (rid < N).astype(jnp.float32)
    hfacc = jnp.zeros((TB, NHID), jnp.float32)
    ssacc = jnp.float32(0.0)
    for s in range(NS):
        hfin = _gru_block(xblk, f_ref, s, Wi_ref[...], bi_ref[...],
                          Wh_ref[...], bh_ref[...])
        hfacc = hfacc + _elu(hfin)
        pred = jnp.dot(hfin, Wss_ref[...],
                       preferred_element_type=jnp.float32) + bss_ref[...]
        dd = (pred - hblk) * maskf
        ssacc = ssacc + jnp.sum(dd * dd)
    acc_ref[0, 0] += ssacc
    gsum_ref[...] += jnp.sum(hfacc * (0.5 * maskf), axis=0, keepdims=True)

    @pl.when(i == TG - 1)
    def _():
        g = gsum_ref[...] * (1.0 / N)
        t1 = jnp.dot(_elu(g), Wd1_ref[...],
                     preferred_element_type=jnp.float32) + bd1_ref[...]
        y_ref[...] = jnp.dot(_elu(t1), Wd2_ref[...],
                             preferred_element_type=jnp.float32) + bd2_ref[...]
        l1_ref[...] = jnp.reshape(
            acc_ref[0, 0] * (1.0 / (NS * N * NIN)), (1, 1))


def _full_spec(shape):
    return pl.BlockSpec(shape, lambda i: tuple(0 for _ in shape))


_enc_in_specs = [
    pl.BlockSpec((TB, NIN), lambda i: (i, 0)),
    _full_spec((NIN, NHID)),
    _full_spec((1, NHID)),
]
_enc_out_specs = [pl.BlockSpec((TB, NHID), lambda i: (i, 0))]
_enc_out_shape = [jax.ShapeDtypeStruct((NP, NHID), jnp.float32)]

_tc_enc = pl.pallas_call(
    _enc_body,
    grid=(TG,),
    in_specs=_enc_in_specs,
    out_specs=_enc_out_specs[0],
    out_shape=_enc_out_shape[0],
)

_tc0_in_specs = [
    pl.BlockSpec((TB, NHID), lambda i: (i, 0)),
    pl.BlockSpec((6, TB, NIN), lambda i: (0, i, 0)),
    pl.BlockSpec((TB, NIN), lambda i: (i, 0)),
    _full_spec((NHID, 3 * NHID)),
    _full_spec((1, 3 * NHID)),
    _full_spec((NHID, 3 * NHID)),
    _full_spec((1, 3 * NHID)),
    _full_spec((NHID, NIN)),
    _full_spec((1, NIN)),
]
_tc0_out_specs = [
    pl.BlockSpec((TB, NHID), lambda i: (i, 0)),
    pl.BlockSpec((1, 1), lambda i: (0, 0)),
]
_tc0_out_shape = [
    jax.ShapeDtypeStruct((NP, NHID), jnp.float32),
    jax.ShapeDtypeStruct((1, 1), jnp.float32),
]
_tc0_scratch = [
    pltpu.SMEM((1, 1), jnp.float32),
]

_tc_layer0 = pl.pallas_call(
    _l0_body,
    grid=(TG,),
    in_specs=_tc0_in_specs,
    out_specs=_tc0_out_specs,
    out_shape=_tc0_out_shape,
    scratch_shapes=_tc0_scratch,
)

_tc1_in_specs = [
    pl.BlockSpec((TB, NHID), lambda i: (i, 0)),
    pl.BlockSpec((6, TB, NHID), lambda i: (0, i, 0)),
    pl.BlockSpec((TB, NIN), lambda i: (i, 0)),
    _full_spec((NHID, 3 * NHID)),
    _full_spec((1, 3 * NHID)),
    _full_spec((NHID, 3 * NHID)),
    _full_spec((1, 3 * NHID)),
    _full_spec((NHID, NIN)),
    _full_spec((1, NIN)),
    _full_spec((NHID, NHID)),
    _full_spec((1, NHID)),
    _full_spec((NHID, 1)),
    _full_spec((1, 1)),
]
_tc1_out_specs = [
    pl.BlockSpec((1, 1), lambda i: (0, 0)),
    pl.BlockSpec((1, 1), lambda i: (0, 0)),
]
_tc1_out_shape = [
    jax.ShapeDtypeStruct((1, 1), jnp.float32),
    jax.ShapeDtypeStruct((1, 1), jnp.float32),
]
_tc1_scratch = [
    pltpu.VMEM((1, NHID), jnp.float32),
    pltpu.SMEM((1, 1), jnp.float32),
]

_tc_layer1 = pl.pallas_call(
    _l1_body,
    grid=(TG,),
    in_specs=_tc1_in_specs,
    out_specs=_tc1_out_specs,
    out_shape=_tc1_out_shape,
    scratch_shapes=_tc1_scratch,
)


def kernel(h, edge_index, W_enc, b_enc, Wi0, Wh0, bi0, bh0, Wss0, bss0,
           Wi1, Wh1, bi1, bh1, Wss1, bss1, Wd1, bd1, Wd2, bd2):
    src = edge_index[0]
    dst = edge_index[1]
    _, dst_sorted = lax.sort_key_val(src, dst, is_stable=True)
    dst_sorted = dst_sorted.astype(jnp.int32)
    deg = jnp.bincount(src, length=N).astype(jnp.int32)
    offsets = (jnp.cumsum(deg) - deg).astype(jnp.int32)
    degp = jnp.pad(deg, (0, NP - N))
    offp = jnp.pad(offsets, (0, NP - N))

    wkey = jax.random.key(42)
    keys = []
    for layer in range(2):
        lk = jax.random.fold_in(wkey, layer)
        for s in range(NS):
            for t in range(LEN - 1):
                keys.append(jax.random.fold_in(lk, s * 97 + t))
    u_all = jax.vmap(lambda k: jax.random.uniform(k, (N,)))(jnp.stack(keys))
    u_all = u_all.reshape(4, LEN - 1, N)
    u_all = jnp.pad(u_all, ((0, 0), (0, 0), (0, NP - N)))
    u_all = u_all.reshape(4, LEN - 1, NW, KJ, KW)

    hp = jnp.pad(h, ((0, NP - N), (0, 0)))
    xp = _tc_enc(hp, W_enc, b_enc.reshape(1, NHID))

    walks1, feats0 = _get_sc_walks()(degp, offp, dst_sorted, u_all, xp)

    bi02 = bi0.reshape(1, 3 * NHID)
    bh02 = bh0.reshape(1, 3 * NHID)
    bss02 = bss0.reshape(1, NIN)
    x1p, l0 = _tc_layer0(xp, feats0, hp, Wi0, bi02, Wh0, bh02,
                         Wss0, bss02)

    (feats1,) = _get_sc_gather()(x1p, walks1)

    bi12 = bi1.reshape(1, 3 * NHID)
    bh12 = bh1.reshape(1, 3 * NHID)
    bss12 = bss1.reshape(1, NIN)
    bd12 = bd1.reshape(1, NHID)
    bd22 = bd2.reshape(1, 1)
    y, l1 = _tc_layer1(x1p, feats1, hp, Wi1, bi12, Wh1, bh12, Wss1, bss12,
                       Wd1, bd12, Wd2, bd22)

    loss = SSW * (l0[0, 0] + l1[0, 0])
    return (y, loss)
